# Initial kernel scaffold; baseline (speedup 1.0000x reference)
import jax
import jax.numpy as jnp
from jax.experimental import pallas as pl


def kernel(x_focal, p_focal, x_neighbor, p_neighbor, edge_attr_neighbor, x_center, x_support, edge_attr_support, p_support):
    raise NotImplementedError("write your pallas kernel here")


# speedup vs baseline: 11.5380x; 11.5380x over previous
import functools
import math
from itertools import permutations

import jax
import jax.numpy as jnp
from jax import lax
from jax.experimental import pallas as pl
from jax.experimental.pallas import tpu as pltpu
from jax.experimental.pallas import tpu_sc as plsc

L = 8
S = 3
D = 3
ND = 16
ED = 8
N = 10000

NC = 2
NSUB = 16
NW = NC * NSUB
SLAB = 320
CHUNKS = SLAB // 16

PERMS = list(permutations(range(S)))
P = len(PERMS)
HALF_PI = math.pi / 2.0

ATAN_C = (
    0.9999994160035334, -0.33330222355322575, 0.1995111089191479,
    -0.13933229393258326, 0.09709350736839653, -0.05688089273448824,
    0.022566826119350198, -0.004257409075516719,
)

R_XSB = 0
R_ESB = R_XSB + 24 * 16
R_XCB = R_ESB + 24 * 8
R_ISUP = R_XCB + 8 * 16
R_LSUP = R_ISUP + L * P * S
R_XS2 = R_LSUP + L * P * S
R_ES2 = R_XS2 + L
R_XC2 = R_ES2 + L
C_ROWS = 1024


def _rsqrt(x):
    i = lax.bitcast_convert_type(x, jnp.int32)
    i = jnp.int32(0x5F3759DF) - (i >> 1)
    y = lax.bitcast_convert_type(i, jnp.float32)
    for _ in range(3):
        y = y * (1.5 - 0.5 * x * y * y)
    return y


def _atan_recip(u):
    u = jnp.maximum(u, 0.0)
    recip = 1.0 / u
    t = jnp.minimum(u, recip)
    t2 = t * t
    f = jnp.full((16,), ATAN_C[-1], jnp.float32)
    for c in ATAN_C[-2::-1]:
        f = f * t2 + c
    r = t * f
    return jnp.where(u >= 1.0, r, HALF_PI - r)


def _sc_body(xn_h, xf_h, en_h, pn_h, pf_h, cst_h, out_h,
             xn_v, xf_v, en_v, pn_v, pf_v, cst, out_v, dxx_v, dee_v):
    wid = lax.axis_index("s") * NC + lax.axis_index("c")
    base = jnp.minimum(wid * SLAB, N - SLAB)
    pltpu.sync_copy(xn_h.at[pl.ds(base * (S * ND), SLAB * S * ND)], xn_v)
    pltpu.sync_copy(xf_h.at[pl.ds(base * ND, SLAB * ND)], xf_v)
    pltpu.sync_copy(en_h.at[pl.ds(base * (S * ED), SLAB * S * ED)], en_v)
    pltpu.sync_copy(pn_h.at[pl.ds(base * (S * D), SLAB * S * D)], pn_v)
    pltpu.sync_copy(pf_h.at[pl.ds(base * D, SLAB * D)], pf_v)
    pltpu.sync_copy(cst_h, cst)

    iota = lax.iota(jnp.int32, 16)
    i48 = iota * (S * ND)
    i24 = iota * (S * ED)
    i16 = iota * ND
    i9 = iota * (S * D)
    i3 = iota * D

    def chunk_body(ci, _):
        col = ci * 16

        xn = [plsc.load_gather(xn_v, [i48 + (col * (S * ND) + r)])
              for r in range(S * ND)]
        en = [plsc.load_gather(en_v, [i24 + (col * (S * ED) + r)])
              for r in range(S * ED)]
        xf = [plsc.load_gather(xf_v, [i16 + (col * ND + r)])
              for r in range(ND)]
        pf = [plsc.load_gather(pf_v, [i3 + (col * D + d)])
              for d in range(D)]
        pn = [[plsc.load_gather(pn_v, [i9 + (col * (S * D) + s * D + d)]) - pf[d]
               for d in range(D)] for s in range(S)]

        xn2 = xn[0] * xn[0]
        for v in xn[1:]:
            xn2 = xn2 + v * v
        en2 = en[0] * en[0]
        for v in en[1:]:
            en2 = en2 + v * v
        xf2 = xf[0] * xf[0]
        for v in xf[1:]:
            xf2 = xf2 + v * v

        n2 = []
        for s in range(S):
            a = pn[s][0] * pn[s][0]
            for d in range(1, D):
                a = a + pn[s][d] * pn[s][d]
            n2.append(a)
        nrm = [n2[s] * _rsqrt(jnp.maximum(n2[s], 1e-30)) for s in range(S)]
        na = [jnp.maximum(nrm[s], 1e-8) for s in range(S)]
        intra_nei = []
        for s in range(S):
            sp = (s - 1) % S
            dsum = pn[sp][0] * pn[s][0]
            for d in range(1, D):
                dsum = dsum + pn[sp][d] * pn[s][d]
            intra_nei.append(dsum / (na[sp] * na[s]))
        len_nei = nrm

        def dxx_body(lt, _):
            accs = None
            for k in range(ND):
                c = cst[pl.ds((R_XSB + lt * ND + k) * 16, 16)]
                if accs is None:
                    accs = [xn[s * ND + k] * c for s in range(S)]
                else:
                    accs = [accs[s] + xn[s * ND + k] * c for s in range(S)]
            for s in range(S):
                dxx_v[pl.ds((s * 24 + lt) * 16, 16)] = accs[s]
            return 0

        lax.fori_loop(0, 24, dxx_body, 0)

        def dee_body(lt, _):
            accs = None
            for k in range(ED):
                c = cst[pl.ds((R_ESB + lt * ED + k) * 16, 16)]
                if accs is None:
                    accs = [en[s * ED + k] * c for s in range(S)]
                else:
                    accs = [accs[s] + en[s * ED + k] * c for s in range(S)]
            for s in range(S):
                dee_v[pl.ds((s * 24 + lt) * 16, 16)] = accs[s]
            return 0

        lax.fori_loop(0, 24, dee_body, 0)

        def l_body(l, _):
            dxx = [[dxx_v[pl.ds((s * 24 + l * S + t) * 16, 16)]
                    for t in range(S)] for s in range(S)]
            dee = [[dee_v[pl.ds((s * 24 + l * S + t) * 16, 16)]
                    for t in range(S)] for s in range(S)]

            best_c = None
            best_ce = None
            best_i = [None] * S
            best_le = [None] * S
            for p, perm in enumerate(PERMS):
                cp = dxx[0][perm[0]] + dxx[1][perm[1]] + dxx[2][perm[2]]
                cep = dee[0][perm[0]] + dee[1][perm[1]] + dee[2][perm[2]]
                isup = [cst[pl.ds((R_ISUP + (l * P + p) * S + s) * 16, 16)]
                        for s in range(S)]
                lsup = [cst[pl.ds((R_LSUP + (l * P + p) * S + s) * 16, 16)]
                        for s in range(S)]
                if p == 0:
                    best_c, best_ce, best_i, best_le = cp, cep, isup, lsup
                else:
                    m = cp > best_c
                    best_c = jnp.where(m, cp, best_c)
                    best_ce = jnp.where(m, cep, best_ce)
                    best_i = [jnp.where(m, isup[s], best_i[s]) for s in range(S)]
                    best_le = [jnp.where(m, lsup[s], best_le[s]) for s in range(S)]

            xs2 = cst[pl.ds((R_XS2 + l) * 16, 16)]
            es2 = cst[pl.ds((R_ES2 + l) * 16, 16)]
            xc2 = cst[pl.ds((R_XC2 + l) * 16, 16)]

            support_sc = _atan_recip(xn2 + xs2 - 2.0 * best_c)

            asum = None
            lsum = None
            for s in range(S):
                da = intra_nei[s] - best_i[s]
                dl = len_nei[s] - best_le[s]
                asum = da * da if asum is None else asum + da * da
                lsum = dl * dl if lsum is None else lsum + dl * dl
            angle_sc = _atan_recip(asum)
            length_sc = _atan_recip(lsum)

            dfc = None
            for k in range(ND):
                c = cst[pl.ds((R_XCB + l * ND + k) * 16, 16)]
                dfc = xf[k] * c if dfc is None else dfc + xf[k] * c
            center_sc = _atan_recip(xf2 + xc2 - 2.0 * dfc)

            edge_sc = _atan_recip(en2 + es2 - 2.0 * best_ce)

            t1 = length_sc - HALF_PI
            t2 = angle_sc - HALF_PI
            t3 = support_sc - HALF_PI
            t4 = center_sc - HALF_PI
            t5 = edge_sc - HALF_PI
            sc = _atan_recip(t1 * t1 + t2 * t2 + t3 * t3 + t4 * t4 + t5 * t5)
            out_v[pl.ds(l * SLAB + col, 16)] = sc
            return 0

        lax.fori_loop(0, L, l_body, 0)
        return 0

    lax.fori_loop(0, CHUNKS, chunk_body, 0)
    for l in range(L):
        pltpu.sync_copy(out_v.at[pl.ds(l * SLAB, SLAB)],
                        out_h.at[pl.ds(l * N + base, SLAB)])


_sc_kernel = functools.partial(
    pl.kernel,
    mesh=plsc.VectorSubcoreMesh(core_axis_name="c", subcore_axis_name="s"),
    compiler_params=pltpu.CompilerParams(needs_layout_passes=False),
    out_type=jax.ShapeDtypeStruct((L * N,), jnp.float32),
    scratch_types=[
        pltpu.VMEM((SLAB * S * ND,), jnp.float32),
        pltpu.VMEM((SLAB * ND,), jnp.float32),
        pltpu.VMEM((SLAB * S * ED,), jnp.float32),
        pltpu.VMEM((SLAB * S * D,), jnp.float32),
        pltpu.VMEM((SLAB * D,), jnp.float32),
        pltpu.VMEM((C_ROWS * 16,), jnp.float32),
        pltpu.VMEM((L * SLAB,), jnp.float32),
        pltpu.VMEM((S * 24 * 16,), jnp.float32),
        pltpu.VMEM((S * 24 * 16,), jnp.float32),
    ],
)(_sc_body)


def kernel(x_focal, p_focal, x_neighbor, p_neighbor, edge_attr_neighbor,
           x_center, x_support, edge_attr_support, p_support):
    f32 = jnp.float32
    xn_h = x_neighbor.astype(f32).reshape(-1)
    xf_h = x_focal.astype(f32).reshape(-1)
    en_h = edge_attr_neighbor.astype(f32).reshape(-1)
    pn_h = p_neighbor.astype(f32).reshape(-1)
    pf_h = p_focal.astype(f32).reshape(-1)

    xs = x_support.astype(f32)
    es = edge_attr_support.astype(f32)
    ps = p_support.astype(f32)
    xc = x_center.astype(f32)[:, 0]

    xs2 = jnp.sum(xs * xs, axis=(1, 2))
    es2 = jnp.sum(es * es, axis=(1, 2))
    xc2 = jnp.sum(xc * xc, axis=1)

    perm_ps = jnp.stack([ps[:, list(p), :] for p in PERMS], axis=1)
    n2 = jnp.sum(perm_ps * perm_ps, axis=-1)
    dots = jnp.sum(jnp.roll(perm_ps, 1, axis=-2) * perm_ps, axis=-1)
    nrm = jnp.sqrt(n2)
    isup = dots / (jnp.maximum(jnp.roll(nrm, 1, axis=-1), 1e-8)
                   * jnp.maximum(nrm, 1e-8))
    lsup = nrm

    rows = jnp.zeros((C_ROWS,), f32)
    rows = rows.at[R_XSB:R_XSB + 24 * 16].set(xs.reshape(-1))
    rows = rows.at[R_ESB:R_ESB + 24 * 8].set(es.reshape(-1))
    rows = rows.at[R_XCB:R_XCB + 8 * 16].set(xc.reshape(-1))
    rows = rows.at[R_ISUP:R_ISUP + L * P * S].set(isup.reshape(-1))
    rows = rows.at[R_LSUP:R_LSUP + L * P * S].set(lsup.reshape(-1))
    rows = rows.at[R_XS2:R_XS2 + L].set(xs2)
    rows = rows.at[R_ES2:R_ES2 + L].set(es2)
    rows = rows.at[R_XC2:R_XC2 + L].set(xc2)
    cst_h = jnp.repeat(rows[:, None], 16, axis=1).reshape(-1)

    out = _sc_kernel(xn_h, xf_h, en_h, pn_h, pf_h, cst_h)
    return out.reshape(L, N)


# speedup vs baseline: 16.8159x; 1.4574x over previous
import functools
import math
from itertools import permutations

import jax
import jax.numpy as jnp
from jax import lax
from jax.experimental import pallas as pl
from jax.experimental.pallas import tpu as pltpu
from jax.experimental.pallas import tpu_sc as plsc

L = 8
S = 3
D = 3
ND = 16
ED = 8
N = 10000

NC = 2
NSUB = 16
NW = NC * NSUB
SLAB = 320
CHUNKS = SLAB // 16

REC = 101
O_XN = 0
O_XF = 48
O_EN = 64
O_PN = 88
O_PF = 97

PERMS = list(permutations(range(S)))
P = len(PERMS)
HALF_PI = math.pi / 2.0

ATAN_C = (
    0.9999994160035334, -0.33330222355322575, 0.1995111089191479,
    -0.13933229393258326, 0.09709350736839653, -0.05688089273448824,
    0.022566826119350198, -0.004257409075516719,
)

R_XSB = 0
R_ESB = R_XSB + 24 * 16
R_XCB = R_ESB + 24 * 8
R_ISUP = R_XCB + 8 * 16
R_LSUP = R_ISUP + L * P * S
R_XS2 = R_LSUP + L * P * S
R_ES2 = R_XS2 + L
R_XC2 = R_ES2 + L
C_ROWS = 1024


def _rsqrt(x):
    i = lax.bitcast_convert_type(x, jnp.int32)
    i = jnp.int32(0x5F3759DF) - (i >> 1)
    y = lax.bitcast_convert_type(i, jnp.float32)
    for _ in range(3):
        y = y * (1.5 - 0.5 * x * y * y)
    return y


def _atan_recip(u):
    u = jnp.maximum(u, 0.0)
    recip = 1.0 / u
    t = jnp.minimum(u, recip)
    t2 = t * t
    f = jnp.full((16,), ATAN_C[-1], jnp.float32)
    for c in ATAN_C[-2::-1]:
        f = f * t2 + c
    r = t * f
    return jnp.where(u >= 1.0, r, HALF_PI - r)


def _sc_body(rec_h, cst_h, out_h, rec_v, cst, out_v, dxx_v, dee_v):
    wid = lax.axis_index("s") * NC + lax.axis_index("c")
    base = jnp.minimum(wid * SLAB, N - SLAB)
    pltpu.sync_copy(rec_h.at[pl.ds(base * REC, SLAB * REC)], rec_v)
    pltpu.sync_copy(cst_h, cst)

    irec = lax.iota(jnp.int32, 16) * REC

    def chunk_body(ci, _):
        col = ci * 16
        cbase = col * REC

        def g(off):
            return plsc.load_gather(rec_v, [irec + (cbase + off)])

        xn = [g(O_XN + r) for r in range(S * ND)]
        en = [g(O_EN + r) for r in range(S * ED)]
        xf = [g(O_XF + r) for r in range(ND)]
        pf = [g(O_PF + d) for d in range(D)]
        pn = [[g(O_PN + s * D + d) - pf[d] for d in range(D)] for s in range(S)]

        xn2 = xn[0] * xn[0]
        for v in xn[1:]:
            xn2 = xn2 + v * v
        en2 = en[0] * en[0]
        for v in en[1:]:
            en2 = en2 + v * v
        xf2 = xf[0] * xf[0]
        for v in xf[1:]:
            xf2 = xf2 + v * v

        n2 = []
        for s in range(S):
            a = pn[s][0] * pn[s][0]
            for d in range(1, D):
                a = a + pn[s][d] * pn[s][d]
            n2.append(a)
        nrm = [n2[s] * _rsqrt(jnp.maximum(n2[s], 1e-30)) for s in range(S)]
        na = [jnp.maximum(nrm[s], 1e-8) for s in range(S)]
        intra_nei = []
        for s in range(S):
            sp = (s - 1) % S
            dsum = pn[sp][0] * pn[s][0]
            for d in range(1, D):
                dsum = dsum + pn[sp][d] * pn[s][d]
            intra_nei.append(dsum / (na[sp] * na[s]))
        len_nei = nrm

        def dxx_body(lt, _):
            accs = None
            for k in range(ND):
                c = cst[pl.ds((R_XSB + lt * ND + k) * 16, 16)]
                if accs is None:
                    accs = [xn[s * ND + k] * c for s in range(S)]
                else:
                    accs = [accs[s] + xn[s * ND + k] * c for s in range(S)]
            for s in range(S):
                dxx_v[pl.ds((s * 24 + lt) * 16, 16)] = accs[s]
            return 0

        lax.fori_loop(0, 24, dxx_body, 0)

        def dee_body(lt, _):
            accs = None
            for k in range(ED):
                c = cst[pl.ds((R_ESB + lt * ED + k) * 16, 16)]
                if accs is None:
                    accs = [en[s * ED + k] * c for s in range(S)]
                else:
                    accs = [accs[s] + en[s * ED + k] * c for s in range(S)]
            for s in range(S):
                dee_v[pl.ds((s * 24 + lt) * 16, 16)] = accs[s]
            return 0

        lax.fori_loop(0, 24, dee_body, 0)

        def l_body(l, _):
            dxx = [[dxx_v[pl.ds((s * 24 + l * S + t) * 16, 16)]
                    for t in range(S)] for s in range(S)]
            dee = [[dee_v[pl.ds((s * 24 + l * S + t) * 16, 16)]
                    for t in range(S)] for s in range(S)]

            best_c = None
            best_ce = None
            best_i = [None] * S
            best_le = [None] * S
            for p, perm in enumerate(PERMS):
                cp = dxx[0][perm[0]] + dxx[1][perm[1]] + dxx[2][perm[2]]
                cep = dee[0][perm[0]] + dee[1][perm[1]] + dee[2][perm[2]]
                isup = [cst[pl.ds((R_ISUP + (l * P + p) * S + s) * 16, 16)]
                        for s in range(S)]
                lsup = [cst[pl.ds((R_LSUP + (l * P + p) * S + s) * 16, 16)]
                        for s in range(S)]
                if p == 0:
                    best_c, best_ce, best_i, best_le = cp, cep, isup, lsup
                else:
                    m = cp > best_c
                    best_c = jnp.where(m, cp, best_c)
                    best_ce = jnp.where(m, cep, best_ce)
                    best_i = [jnp.where(m, isup[s], best_i[s]) for s in range(S)]
                    best_le = [jnp.where(m, lsup[s], best_le[s]) for s in range(S)]

            xs2 = cst[pl.ds((R_XS2 + l) * 16, 16)]
            es2 = cst[pl.ds((R_ES2 + l) * 16, 16)]
            xc2 = cst[pl.ds((R_XC2 + l) * 16, 16)]

            support_sc = _atan_recip(xn2 + xs2 - 2.0 * best_c)

            asum = None
            lsum = None
            for s in range(S):
                da = intra_nei[s] - best_i[s]
                dl = len_nei[s] - best_le[s]
                asum = da * da if asum is None else asum + da * da
                lsum = dl * dl if lsum is None else lsum + dl * dl
            angle_sc = _atan_recip(asum)
            length_sc = _atan_recip(lsum)

            dfc = None
            for k in range(ND):
                c = cst[pl.ds((R_XCB + l * ND + k) * 16, 16)]
                dfc = xf[k] * c if dfc is None else dfc + xf[k] * c
            center_sc = _atan_recip(xf2 + xc2 - 2.0 * dfc)

            edge_sc = _atan_recip(en2 + es2 - 2.0 * best_ce)

            t1 = length_sc - HALF_PI
            t2 = angle_sc - HALF_PI
            t3 = support_sc - HALF_PI
            t4 = center_sc - HALF_PI
            t5 = edge_sc - HALF_PI
            sc = _atan_recip(t1 * t1 + t2 * t2 + t3 * t3 + t4 * t4 + t5 * t5)
            out_v[pl.ds(l * SLAB + col, 16)] = sc
            return 0

        lax.fori_loop(0, L, l_body, 0)
        return 0

    lax.fori_loop(0, CHUNKS, chunk_body, 0)
    for l in range(L):
        pltpu.sync_copy(out_v.at[pl.ds(l * SLAB, SLAB)],
                        out_h.at[pl.ds(l * N + base, SLAB)])


_sc_kernel = functools.partial(
    pl.kernel,
    mesh=plsc.VectorSubcoreMesh(core_axis_name="c", subcore_axis_name="s"),
    compiler_params=pltpu.CompilerParams(needs_layout_passes=False),
    out_type=jax.ShapeDtypeStruct((L * N,), jnp.float32),
    scratch_types=[
        pltpu.VMEM((SLAB * REC,), jnp.float32),
        pltpu.VMEM((C_ROWS * 16,), jnp.float32),
        pltpu.VMEM((L * SLAB,), jnp.float32),
        pltpu.VMEM((S * 24 * 16,), jnp.float32),
        pltpu.VMEM((S * 24 * 16,), jnp.float32),
    ],
)(_sc_body)


def kernel(x_focal, p_focal, x_neighbor, p_neighbor, edge_attr_neighbor,
           x_center, x_support, edge_attr_support, p_support):
    f32 = jnp.float32
    rec_h = jnp.concatenate([
        x_neighbor.astype(f32).reshape(N, S * ND),
        x_focal.astype(f32),
        edge_attr_neighbor.astype(f32).reshape(N, S * ED),
        p_neighbor.astype(f32).reshape(N, S * D),
        p_focal.astype(f32),
        jnp.zeros((N, REC - 100), f32),
    ], axis=1).reshape(-1)

    xs = x_support.astype(f32)
    es = edge_attr_support.astype(f32)
    ps = p_support.astype(f32)
    xc = x_center.astype(f32)[:, 0]

    xs2 = jnp.sum(xs * xs, axis=(1, 2))
    es2 = jnp.sum(es * es, axis=(1, 2))
    xc2 = jnp.sum(xc * xc, axis=1)

    perm_ps = jnp.stack([ps[:, list(p), :] for p in PERMS], axis=1)
    n2 = jnp.sum(perm_ps * perm_ps, axis=-1)
    dots = jnp.sum(jnp.roll(perm_ps, 1, axis=-2) * perm_ps, axis=-1)
    nrm = jnp.sqrt(n2)
    isup = dots / (jnp.maximum(jnp.roll(nrm, 1, axis=-1), 1e-8)
                   * jnp.maximum(nrm, 1e-8))
    lsup = nrm

    rows = jnp.zeros((C_ROWS,), f32)
    rows = rows.at[R_XSB:R_XSB + 24 * 16].set(xs.reshape(-1))
    rows = rows.at[R_ESB:R_ESB + 24 * 8].set(es.reshape(-1))
    rows = rows.at[R_XCB:R_XCB + 8 * 16].set(xc.reshape(-1))
    rows = rows.at[R_ISUP:R_ISUP + L * P * S].set(isup.reshape(-1))
    rows = rows.at[R_LSUP:R_LSUP + L * P * S].set(lsup.reshape(-1))
    rows = rows.at[R_XS2:R_XS2 + L].set(xs2)
    rows = rows.at[R_ES2:R_ES2 + L].set(es2)
    rows = rows.at[R_XC2:R_XC2 + L].set(xc2)
    cst_h = jnp.repeat(rows[:, None], 16, axis=1).reshape(-1)

    out = _sc_kernel(rec_h, cst_h)
    return out.reshape(L, N)
